# baseline (speedup 1.0000x reference)
import functools

import jax
import jax.numpy as jnp
from jax.experimental import pallas as pl


def _g_kernel(feat_ref, w1_ref, w2_ref, g_ref):
    w12 = jnp.dot(w1_ref[...], w2_ref[...], preferred_element_type=jnp.float32)
    g_ref[...] = jnp.dot(feat_ref[...], w12, preferred_element_type=jnp.float32)


def _spmm_kernel(a_ref, x_ref, y_ref):
    y_ref[...] = jnp.dot(a_ref[...], x_ref[...], preferred_element_type=jnp.float32)


@functools.partial(jax.jit, static_argnames=("bm",))
def _row_stream_matmul(adj, x, bm):
    n, f = adj.shape[0], x.shape[1]
    return pl.pallas_call(
        _spmm_kernel,
        grid=(n // bm,),
        in_specs=[
            pl.BlockSpec((bm, n), lambda i: (i, 0)),
            pl.BlockSpec((n, f), lambda i: (0, 0)),
        ],
        out_specs=pl.BlockSpec((bm, f), lambda i: (i, 0)),
        out_shape=jax.ShapeDtypeStruct((n, f), jnp.float32),
    )(adj, x)


def kernel(feat, adj, W1, W2):
    n = adj.shape[0]
    f = W2.shape[1]
    g = pl.pallas_call(
        _g_kernel,
        out_shape=jax.ShapeDtypeStruct((n, f), jnp.float32),
    )(feat, W1, W2)
    y = _row_stream_matmul(adj, g, bm=200)
    return _row_stream_matmul(adj, y, bm=200)


# speedup vs baseline: 1.0057x; 1.0057x over previous
import functools

import jax
import jax.numpy as jnp
from jax.experimental import pallas as pl
from jax.experimental.pallas import tpu as pltpu

_BM = 200


def _fused_kernel(feat_ref, w1_ref, w2_ref, a_ref, out_ref, xbuf, ybuf):
    p = pl.program_id(0)
    i = pl.program_id(1)

    @pl.when((p == 0) & (i == 0))
    def _prologue():
        w12 = jnp.dot(w1_ref[...], w2_ref[...], preferred_element_type=jnp.float32)
        xbuf[...] = jnp.dot(feat_ref[...], w12, preferred_element_type=jnp.float32)

    @pl.when(p == 0)
    def _pass1():
        ybuf[pl.ds(i * _BM, _BM), :] = jnp.dot(
            a_ref[...], xbuf[...], preferred_element_type=jnp.float32)

    @pl.when(p == 1)
    def _pass2():
        out_ref[...] = jnp.dot(
            a_ref[...], ybuf[...], preferred_element_type=jnp.float32)


@jax.jit
def kernel(feat, adj, W1, W2):
    n = adj.shape[0]
    f = W2.shape[1]
    return pl.pallas_call(
        _fused_kernel,
        grid=(2, n // _BM),
        in_specs=[
            pl.BlockSpec(feat.shape, lambda p, i: (0, 0)),
            pl.BlockSpec(W1.shape, lambda p, i: (0, 0)),
            pl.BlockSpec(W2.shape, lambda p, i: (0, 0)),
            pl.BlockSpec((_BM, n), lambda p, i: (i, 0)),
        ],
        out_specs=pl.BlockSpec((_BM, f), lambda p, i: (i, 0)),
        out_shape=jax.ShapeDtypeStruct((n, f), jnp.float32),
        scratch_shapes=[
            pltpu.VMEM((n, f), jnp.float32),
            pltpu.VMEM((n, f), jnp.float32),
        ],
    )(feat, W1, W2, adj)


# speedup vs baseline: 1.0162x; 1.0105x over previous
import functools

import jax
import jax.numpy as jnp
from jax.experimental import pallas as pl
from jax.experimental.pallas import tpu as pltpu

_BM = 400


def _fused_kernel(feat_ref, w1_ref, w2_ref, a_ref, out_ref, xbuf, ybuf):
    p = pl.program_id(0)
    i = pl.program_id(1)

    @pl.when((p == 0) & (i == 0))
    def _prologue():
        w12 = jnp.dot(w1_ref[...], w2_ref[...], preferred_element_type=jnp.float32)
        xbuf[...] = jnp.dot(feat_ref[...], w12, preferred_element_type=jnp.float32)

    @pl.when(p == 0)
    def _pass1():
        ybuf[pl.ds(i * _BM, _BM), :] = jnp.dot(
            a_ref[...], xbuf[...], preferred_element_type=jnp.float32)

    @pl.when(p == 1)
    def _pass2():
        out_ref[...] = jnp.dot(
            a_ref[...], ybuf[...], preferred_element_type=jnp.float32)


@jax.jit
def kernel(feat, adj, W1, W2):
    n = adj.shape[0]
    f = W2.shape[1]
    return pl.pallas_call(
        _fused_kernel,
        grid=(2, n // _BM),
        in_specs=[
            pl.BlockSpec(feat.shape, lambda p, i: (0, 0)),
            pl.BlockSpec(W1.shape, lambda p, i: (0, 0)),
            pl.BlockSpec(W2.shape, lambda p, i: (0, 0)),
            pl.BlockSpec((_BM, n), lambda p, i: (i, 0)),
        ],
        out_specs=pl.BlockSpec((_BM, f), lambda p, i: (i, 0)),
        out_shape=jax.ShapeDtypeStruct((n, f), jnp.float32),
        scratch_shapes=[
            pltpu.VMEM((n, f), jnp.float32),
            pltpu.VMEM((n, f), jnp.float32),
        ],
    )(feat, W1, W2, adj)
